# baseline (speedup 1.0000x reference)
import functools

import jax
import jax.numpy as jnp
from jax import lax
from jax.experimental import pallas as pl
from jax.experimental.pallas import tpu as pltpu
from jax.experimental.pallas import tpu_sc as plsc

_N, _D, _H, _GH, _E = 2048, 768, 128, 64, 16
_S = 2 * _N
_SB = 128
_NSB = _S // _SB
_CT = 256
_NCT = _S // _CT
_BALANCE_COEF = 0.01


def _gate_sort_body(x_ref, gw1_ref, gb1_ref, gw2_ref, gb2_ref,
                    pos_ref, ww_ref, se_ref,
                    usage_ref, loss_ref):
    x = x_ref[...]
    gh = jnp.maximum(
        jnp.dot(x, gw1_ref[...], preferred_element_type=jnp.float32)
        + gb1_ref[...], 0.0)
    logits = (jnp.dot(gh, gw2_ref[...], preferred_element_type=jnp.float32)
              + gb2_ref[...])
    eid = jax.lax.broadcasted_iota(jnp.int32, logits.shape, 1)
    l1 = jnp.max(logits, axis=1, keepdims=True)
    i1 = jnp.min(jnp.where(logits == l1, eid, _E), axis=1, keepdims=True)
    m1 = eid == i1
    masked = jnp.where(m1, jnp.float32(-1e30), logits)
    l2 = jnp.max(masked, axis=1, keepdims=True)
    i2 = jnp.min(jnp.where(masked == l2, eid, _E), axis=1, keepdims=True)
    m2 = eid == i2
    wa = 1.0 / (1.0 + jnp.exp(l2 - l1))
    onehot = jnp.concatenate(
        [m1.astype(jnp.float32), m2.astype(jnp.float32)], axis=0)
    wcol = jnp.concatenate([wa, 1.0 - wa], axis=0)
    ww_ref[...] = jnp.broadcast_to(wcol, (_S, 128))

    counts = jnp.sum(onehot, axis=0, keepdims=True)
    usage = counts * (1.0 / _N)
    usage_ref[...] = usage
    loss_ref[...] = (jnp.mean((usage - 1.0 / _E) ** 2)
                     * _BALANCE_COEF).reshape(1, 1)

    ue = jax.lax.broadcasted_iota(jnp.int32, (_E, _E), 0)
    ve = jax.lax.broadcasted_iota(jnp.int32, (_E, _E), 1)
    tri_lt = (ue < ve).astype(jnp.float32)
    starts = jnp.dot(counts, tri_lt, preferred_element_type=jnp.float32)
    ends = starts + counts
    se_ref[...] = jnp.concatenate(
        [starts, ends], axis=0).astype(jnp.int32)

    ub = jax.lax.broadcasted_iota(jnp.int32, (_SB, _SB), 0)
    vb = jax.lax.broadcasted_iota(jnp.int32, (_SB, _SB), 1)
    ltb = (vb < ub).astype(jnp.bfloat16)
    within = [None] * _NSB
    btot = [None] * _NSB
    for b in range(_NSB):
        ob = onehot[b * _SB:(b + 1) * _SB, :]
        wb = jnp.dot(ltb, ob.astype(jnp.bfloat16),
                     preferred_element_type=jnp.float32)
        within[b] = wb
        btot[b] = (wb + ob)[_SB - 1:_SB, :]
    btots = jnp.concatenate(btot, axis=0)
    u3 = jax.lax.broadcasted_iota(jnp.int32, (_NSB, _NSB), 0)
    v3 = jax.lax.broadcasted_iota(jnp.int32, (_NSB, _NSB), 1)
    lt3 = (v3 < u3).astype(jnp.float32)
    bpref = jnp.dot(lt3, btots, preferred_element_type=jnp.float32)
    rank = jnp.concatenate(
        [within[b] + bpref[b:b + 1, :] for b in range(_NSB)], axis=0)
    posmat = (rank + starts) * onehot
    ones_e = jnp.ones((_E, 1), jnp.float32)
    pos = jnp.dot(posmat, ones_e, preferred_element_type=jnp.float32)
    pos_ref[...] = pos.astype(jnp.int32)


def _mk_scatter():
    mesh = plsc.VectorSubcoreMesh(core_axis_name="c", subcore_axis_name="s")

    @functools.partial(
        pl.kernel, mesh=mesh,
        out_type=(
            jax.ShapeDtypeStruct((_S, _D), jnp.float32),
            jax.ShapeDtypeStruct((_S, 128), jnp.float32),
        ),
        scratch_types=[
            pltpu.VMEM((_SB,), jnp.int32),
            pltpu.VMEM((_SB, _D), jnp.float32),
            pltpu.VMEM((_SB, 128), jnp.float32),
            pltpu.SemaphoreType.DMA,
            pltpu.SemaphoreType.DMA,
        ],
    )
    def scatter_k(x_hbm, pos_hbm, ww_hbm, xs_hbm, ws_hbm,
                  idx_v, rows_v, wrow_v, sem, sem2):
        w = lax.axis_index("s") * 2 + lax.axis_index("c")
        tok0 = (w % 16) * _SB
        pltpu.sync_copy(pos_hbm.at[pl.ds(w * _SB, _SB)], idx_v)
        pltpu.sync_copy(x_hbm.at[pl.ds(tok0, _SB)], rows_v)
        pltpu.sync_copy(ww_hbm.at[pl.ds(w * _SB, _SB)], wrow_v)
        cpx = pltpu.async_copy(rows_v, xs_hbm.at[idx_v], sem)
        cpw = pltpu.async_copy(wrow_v, ws_hbm.at[idx_v], sem2)
        cpx.wait()
        cpw.wait()

    return scatter_k


def _group_body(xs_ref, ws_ref, se_ref, w1_ref, b1_ref, w2_ref, b2_ref,
                w3_ref, b3_ref, ys_ref, acc, w1c, w2c, w3c):
    i = pl.program_id(0)

    @pl.when(i == 0)
    def _():
        for e in range(_E):
            w1c[:, e * _H:(e + 1) * _H] = w1_ref[e].astype(jnp.bfloat16)
            w2c[e] = w2_ref[e].astype(jnp.bfloat16)
            w3c[e * _H:(e + 1) * _H, :] = w3_ref[e].astype(jnp.bfloat16)

    base = i * _CT
    xt = xs_ref[...].astype(jnp.bfloat16)
    p = base + jax.lax.broadcasted_iota(jnp.int32, (_CT, 1), 0)
    acc[...] = jnp.zeros((_CT, _D), jnp.float32)
    for e in range(_E):
        s_e = se_ref[0, e]
        e_e = se_ref[1, e]

        @pl.when(jnp.logical_and(e_e > base, s_e < base + _CT))
        def _(e=e, s_e=s_e, e_e=e_e):
            h1e = jnp.maximum(
                jnp.dot(xt, w1c[:, e * _H:(e + 1) * _H],
                        preferred_element_type=jnp.float32)
                + b1_ref[:, e * _H:(e + 1) * _H], 0.0)
            h2e = jnp.maximum(
                jnp.dot(h1e.astype(jnp.bfloat16), w2c[e],
                        preferred_element_type=jnp.float32)
                + b2_ref[:, e * _H:(e + 1) * _H], 0.0)
            mcol = jnp.logical_and(p >= s_e, p < e_e).astype(jnp.float32)
            acc[...] += (jnp.dot((h2e * mcol).astype(jnp.bfloat16),
                                 w3c[e * _H:(e + 1) * _H, :],
                                 preferred_element_type=jnp.float32)
                         + mcol * b3_ref[e])
    wsm = ws_ref[...]
    lid = jax.lax.broadcasted_iota(jnp.int32, wsm.shape, 1)
    wcol = jnp.sum(jnp.where(lid == 0, wsm, 0.0), axis=1, keepdims=True)
    ys_ref[...] = acc[...] * wcol


def _mk_combine():
    mesh = plsc.VectorSubcoreMesh(core_axis_name="c", subcore_axis_name="s")
    TB = _N // 32

    @functools.partial(
        pl.kernel, mesh=mesh,
        out_type=jax.ShapeDtypeStruct((_N, _D), jnp.float32),
        scratch_types=[
            pltpu.VMEM((TB,), jnp.int32),
            pltpu.VMEM((TB,), jnp.int32),
            pltpu.VMEM((TB, _D), jnp.float32),
            pltpu.VMEM((TB, _D), jnp.float32),
            pltpu.SemaphoreType.DMA,
            pltpu.SemaphoreType.DMA,
        ],
    )
    def combine_k(pos_hbm, ys_hbm, out_hbm,
                  idx0, idx1, rows0, rows1, sem0, sem1):
        w = lax.axis_index("s") * 2 + lax.axis_index("c")
        t0 = w * TB
        pltpu.sync_copy(pos_hbm.at[pl.ds(t0, TB)], idx0)
        pltpu.sync_copy(pos_hbm.at[pl.ds(_N + t0, TB)], idx1)
        cp0 = pltpu.async_copy(ys_hbm.at[idx0], rows0, sem0)
        cp1 = pltpu.async_copy(ys_hbm.at[idx1], rows1, sem1)
        cp0.wait()
        cp1.wait()
        for t in range(TB):
            def body(ch, carry, t=t):
                a0 = rows0[t, pl.ds(ch * 16, 16)]
                a1 = rows1[t, pl.ds(ch * 16, 16)]
                rows0[t, pl.ds(ch * 16, 16)] = a0 + a1
                return carry
            lax.fori_loop(0, _D // 16, body, 0)
        pltpu.sync_copy(rows0, out_hbm.at[pl.ds(t0, TB)])

    return combine_k


def kernel(x, gate_W1, gate_b1, gate_W2, gate_b2, W1, b1, W2, b2, W3, b3):
    pos, ww, se, usage, loss = pl.pallas_call(
        _gate_sort_body,
        out_shape=(
            jax.ShapeDtypeStruct((_S, 1), jnp.int32),
            jax.ShapeDtypeStruct((_S, 128), jnp.float32),
            jax.ShapeDtypeStruct((2, _E), jnp.int32),
            jax.ShapeDtypeStruct((1, _E), jnp.float32),
            jax.ShapeDtypeStruct((1, 1), jnp.float32),
        ),
    )(x, gate_W1, gate_b1.reshape(1, _GH), gate_W2, gate_b2.reshape(1, _E))

    pos_flat = pos.reshape(_S)
    xs, ws = _mk_scatter()(x, pos_flat, ww)

    ys = pl.pallas_call(
        _group_body,
        grid=(_NCT,),
        in_specs=[
            pl.BlockSpec((_CT, _D), lambda i: (i, 0)),
            pl.BlockSpec((_CT, 128), lambda i: (i, 0)),
            pl.BlockSpec(memory_space=pltpu.SMEM),
            pl.BlockSpec((_E, _D, _H), lambda i: (0, 0, 0)),
            pl.BlockSpec((1, _E * _H), lambda i: (0, 0)),
            pl.BlockSpec((_E, _H, _H), lambda i: (0, 0, 0)),
            pl.BlockSpec((1, _E * _H), lambda i: (0, 0)),
            pl.BlockSpec((_E, _H, _D), lambda i: (0, 0, 0)),
            pl.BlockSpec((_E, _D), lambda i: (0, 0)),
        ],
        out_specs=pl.BlockSpec((_CT, _D), lambda i: (i, 0)),
        out_shape=jax.ShapeDtypeStruct((_S, _D), jnp.float32),
        scratch_shapes=[
            pltpu.VMEM((_CT, _D), jnp.float32),
            pltpu.VMEM((_D, _E * _H), jnp.bfloat16),
            pltpu.VMEM((_E, _H, _H), jnp.bfloat16),
            pltpu.VMEM((_E * _H, _D), jnp.bfloat16),
        ],
    )(xs, ws, se, W1, b1.reshape(1, _E * _H), W2,
      b2.reshape(1, _E * _H), W3, b3)

    out = (jnp.take(ys, pos_flat[:_N], axis=0)
           + jnp.take(ys, pos_flat[_N:], axis=0))
    return out, loss[0, 0], usage.reshape(_E)
